# baseline (speedup 1.0000x reference)
import jax
import jax.numpy as jnp
from jax import lax
from jax.experimental import pallas as pl

_TEMPERATURE = 0.2
_THR = 0.5


def _wta_kernel(x_ref, idx_ref, w_ref, gated_ref):
    x = x_ref[0]
    L2, K2 = x.shape
    K = K2 // 2
    row_iota = lax.broadcasted_iota(jnp.int32, (L2, K2), 0)
    tmin = jnp.min(jnp.where(x > _THR, row_iota, jnp.int32(L2)),
                   axis=0, keepdims=True)
    t_star = jnp.min(tmin)
    has_any = t_star < L2
    lane = lax.broadcasted_iota(jnp.int32, (1, K2), 1)
    k2_star = jnp.min(jnp.where(tmin == t_star, lane, jnp.int32(K2)))
    k_star = lax.rem(k2_star, K)

    def _fallback():
        total = jnp.sum(x, axis=0, keepdims=True)
        tot = total[:, :K] + total[:, K:]
        maxv = jnp.max(tot)
        ki = lax.broadcasted_iota(jnp.int32, (1, K), 1)
        return jnp.min(jnp.where(tot == maxv, ki, jnp.int32(K)))

    idx = lax.cond(has_any, lambda: k_star, _fallback)
    ki64 = lax.broadcasted_iota(jnp.int32, (1, K), 1)
    w_ref[0] = (ki64 == idx).astype(x.dtype)
    wide_mask = lax.bitwise_and(lane, jnp.int32(K - 1)) == idx
    gated_ref[0] = jnp.where(wide_mask, x, jnp.zeros_like(x))
    idx_ref[...] = jnp.broadcast_to(idx, (1, 1, 1))


@jax.jit
def kernel(spikes):
    B, L, K = spikes.shape
    L2, K2 = L // 2, K * 2
    sv = spikes.reshape(B, L2, K2)
    idx3, w3, gated = pl.pallas_call(
        _wta_kernel,
        grid=(B,),
        in_specs=[pl.BlockSpec((1, L2, K2), lambda b: (b, 0, 0))],
        out_specs=[
            pl.BlockSpec((1, 1, 1), lambda b: (b, 0, 0)),
            pl.BlockSpec((1, 1, K), lambda b: (b, 0, 0)),
            pl.BlockSpec((1, L2, K2), lambda b: (b, 0, 0)),
        ],
        out_shape=[
            jax.ShapeDtypeStruct((B, 1, 1), jnp.int32),
            jax.ShapeDtypeStruct((B, 1, K), spikes.dtype),
            jax.ShapeDtypeStruct((B, L2, K2), spikes.dtype),
        ],
    )(sv)
    return idx3.reshape(B), w3.reshape(B, K), gated.reshape(B, L, K)


# speedup vs baseline: 1.4767x; 1.4767x over previous
import jax
import jax.numpy as jnp
from jax import lax
from jax.experimental import pallas as pl

_TEMPERATURE = 0.2
_THR = 0.5
_SB = 4


def _wta_kernel(x_ref, idx_ref, w_ref, gated_ref):
    x = x_ref[...]
    SB, L, K = x.shape
    row_iota = lax.broadcasted_iota(jnp.int32, (SB, L, K), 1)
    tmin = jnp.min(jnp.where(x > _THR, row_iota, jnp.int32(L)),
                   axis=1, keepdims=True)
    t_star = jnp.min(tmin, axis=2, keepdims=True)
    has_any = t_star < L
    lane = lax.broadcasted_iota(jnp.int32, (SB, 1, K), 2)
    k_star = jnp.min(jnp.where(tmin == t_star, lane, jnp.int32(K)),
                     axis=2, keepdims=True)

    def _with_fallback():
        total = jnp.sum(x, axis=1, keepdims=True)
        maxv = jnp.max(total, axis=2, keepdims=True)
        k_fb = jnp.min(jnp.where(total == maxv, lane, jnp.int32(K)),
                       axis=2, keepdims=True)
        return jnp.where(has_any, k_star, k_fb)

    idx = lax.cond(jnp.all(has_any), lambda: k_star, _with_fallback)
    w = (lane == idx).astype(x.dtype)
    w_ref[...] = w
    gated_ref[...] = x * w
    idx_ref[...] = idx


@jax.jit
def kernel(spikes):
    B, L, K = spikes.shape
    sb = _SB if B % _SB == 0 else 1
    nb = B // sb
    idx3, w3, gated = pl.pallas_call(
        _wta_kernel,
        grid=(nb,),
        in_specs=[pl.BlockSpec((sb, L, K), lambda b: (b, 0, 0))],
        out_specs=[
            pl.BlockSpec((sb, 1, 1), lambda b: (b, 0, 0)),
            pl.BlockSpec((sb, 1, K), lambda b: (b, 0, 0)),
            pl.BlockSpec((sb, L, K), lambda b: (b, 0, 0)),
        ],
        out_shape=[
            jax.ShapeDtypeStruct((B, 1, 1), jnp.int32),
            jax.ShapeDtypeStruct((B, 1, K), spikes.dtype),
            jax.ShapeDtypeStruct((B, L, K), spikes.dtype),
        ],
    )(spikes)
    return idx3.reshape(B), w3.reshape(B, K), gated
